# Initial kernel scaffold; baseline (speedup 1.0000x reference)
import jax
import jax.numpy as jnp
from jax.experimental import pallas as pl


def kernel(x, edge_index, edge_attr, batch, params):
    raise NotImplementedError("write your pallas kernel here")


# speedup vs baseline: 1.0856x; 1.0856x over previous
import functools

import jax
import jax.numpy as jnp
from jax.experimental import pallas as pl

N = 10000
E = 160000
H = 256

_NB = 1000
_EB = 1000


def _mm_bias_body(h_ref, w_ref, b_ref, o_ref):
    o_ref[...] = (
        jnp.dot(h_ref[...], w_ref[...], preferred_element_type=jnp.float32)
        + b_ref[...]
    )


def _mm_bias(h, w, b):
    m, k = h.shape
    n = w.shape[1]
    return pl.pallas_call(
        _mm_bias_body,
        grid=(m // _NB,),
        in_specs=[
            pl.BlockSpec((_NB, k), lambda i: (i, 0)),
            pl.BlockSpec((k, n), lambda i: (0, 0)),
            pl.BlockSpec((1, n), lambda i: (0, 0)),
        ],
        out_specs=pl.BlockSpec((_NB, n), lambda i: (i, 0)),
        out_shape=jax.ShapeDtypeStruct((m, n), jnp.float32),
    )(h, w, b.reshape(1, n))


def _gnn_update_body(h_ref, agg_ref, gate_ref, wa_ref, wb_ref, b_ref, o_ref):
    z = (
        jnp.dot(h_ref[...], wa_ref[...], preferred_element_type=jnp.float32)
        + jnp.dot(agg_ref[...], wb_ref[...], preferred_element_type=jnp.float32)
        + b_ref[...]
    )
    o_ref[...] = jnp.maximum(z, 0.0) * gate_ref[...]


def _gnn_update(h, agg, gate, wa, wb, b):
    return pl.pallas_call(
        _gnn_update_body,
        grid=(N // _NB,),
        in_specs=[
            pl.BlockSpec((_NB, H), lambda i: (i, 0)),
            pl.BlockSpec((_NB, H), lambda i: (i, 0)),
            pl.BlockSpec((_NB, 1), lambda i: (i, 0)),
            pl.BlockSpec((H, H), lambda i: (0, 0)),
            pl.BlockSpec((H, H), lambda i: (0, 0)),
            pl.BlockSpec((1, H), lambda i: (0, 0)),
        ],
        out_specs=pl.BlockSpec((_NB, H), lambda i: (i, 0)),
        out_shape=jax.ShapeDtypeStruct((N, H), jnp.float32),
    )(h, agg, gate, wa, wb, b.reshape(1, H))


def _epilogue_body(h_ref, pv_ref, deg_ref, wr_ref, wc_ref, wv_ref, b1_ref,
                   cw1_ref, cb1_ref, cw2_ref, cb2_ref,
                   hf_ref, an_ref, bn_ref, pr_ref, vio_ref):
    pv = pv_ref[...]
    vio = jnp.maximum(deg_ref[...] - pv, 0.0)
    vio_ref[...] = vio
    hf = h_ref[...] / (1.0 + vio)
    hf_ref[...] = hf
    wv = wv_ref[...]
    an_ref[...] = (
        jnp.dot(hf, wr_ref[...], preferred_element_type=jnp.float32)
        + pv * wv[0:1, :]
    )
    bn_ref[...] = (
        jnp.dot(hf, wc_ref[...], preferred_element_type=jnp.float32)
        + pv * wv[1:2, :] + b1_ref[...]
    )
    p1 = jnp.maximum(
        jnp.dot(hf, cw1_ref[...], preferred_element_type=jnp.float32)
        + cb1_ref[...], 0.0)
    pr_ref[...] = (
        jnp.dot(p1, cw2_ref[...], preferred_element_type=jnp.float32)
        + cb2_ref[...]
    )


def _epilogue(h, pv_f, degree, wr, wc, wv, b1, cw1, cb1, cw2, cb2):
    out_shapes = (
        jax.ShapeDtypeStruct((N, H), jnp.float32),
        jax.ShapeDtypeStruct((N, 128), jnp.float32),
        jax.ShapeDtypeStruct((N, 128), jnp.float32),
        jax.ShapeDtypeStruct((N, 32), jnp.float32),
        jax.ShapeDtypeStruct((N, 1), jnp.float32),
    )
    return pl.pallas_call(
        _epilogue_body,
        grid=(N // _NB,),
        in_specs=[
            pl.BlockSpec((_NB, H), lambda i: (i, 0)),
            pl.BlockSpec((_NB, 1), lambda i: (i, 0)),
            pl.BlockSpec((_NB, 1), lambda i: (i, 0)),
            pl.BlockSpec((H, 128), lambda i: (0, 0)),
            pl.BlockSpec((H, 128), lambda i: (0, 0)),
            pl.BlockSpec((2, 128), lambda i: (0, 0)),
            pl.BlockSpec((1, 128), lambda i: (0, 0)),
            pl.BlockSpec((H, 128), lambda i: (0, 0)),
            pl.BlockSpec((1, 128), lambda i: (0, 0)),
            pl.BlockSpec((128, 32), lambda i: (0, 0)),
            pl.BlockSpec((1, 32), lambda i: (0, 0)),
        ],
        out_specs=[
            pl.BlockSpec((_NB, H), lambda i: (i, 0)),
            pl.BlockSpec((_NB, 128), lambda i: (i, 0)),
            pl.BlockSpec((_NB, 128), lambda i: (i, 0)),
            pl.BlockSpec((_NB, 32), lambda i: (i, 0)),
            pl.BlockSpec((_NB, 1), lambda i: (i, 0)),
        ],
        out_shape=out_shapes,
    )(h, pv_f, degree, wr, wc, wv, b1.reshape(1, 128),
      cw1, cb1.reshape(1, 128), cw2, cb2.reshape(1, 32))


def _edge_cls_body(ar_ref, bc_ref, meta_ref, w2_ref, b2_ref, w3_ref, b3_ref,
                   o_ref):
    f1 = jnp.maximum(ar_ref[...] + bc_ref[...], 0.0)
    z = jnp.maximum(
        jnp.dot(f1, w2_ref[...], preferred_element_type=jnp.float32)
        + b2_ref[...], 0.0)
    lg = jnp.dot(z, w3_ref[...], preferred_element_type=jnp.float32) + b3_ref[...]
    meta = meta_ref[...]
    at1 = meta[:, 0:1]
    at2 = meta[:, 1:2]
    pv1 = meta[:, 2:3]
    pv2 = meta[:, 3:4]
    noble = (at1 == 4.0) | (at1 == 5.0) | (at2 == 4.0) | (at2 == 5.0)
    m2 = (pv1 <= 2.0) | (pv2 <= 2.0)
    m3 = (pv1 <= 1.0) | (pv2 <= 1.0)
    ci = jax.lax.broadcasted_iota(jnp.int32, lg.shape, 1)
    pen = jnp.where((ci >= 1) & noble, -100.0, 0.0)
    pen = pen + jnp.where((ci == 2) & m2, -50.0, 0.0)
    pen = pen + jnp.where((ci >= 1) & m3, -50.0, 0.0)
    o_ref[...] = lg + pen


def _edge_cls(arow, bcol, meta, w2, b2, w3, b3):
    return pl.pallas_call(
        _edge_cls_body,
        grid=(E // _EB,),
        in_specs=[
            pl.BlockSpec((_EB, 128), lambda i: (i, 0)),
            pl.BlockSpec((_EB, 128), lambda i: (i, 0)),
            pl.BlockSpec((_EB, 4), lambda i: (i, 0)),
            pl.BlockSpec((128, 64), lambda i: (0, 0)),
            pl.BlockSpec((1, 64), lambda i: (0, 0)),
            pl.BlockSpec((64, 4), lambda i: (0, 0)),
            pl.BlockSpec((1, 4), lambda i: (0, 0)),
        ],
        out_specs=pl.BlockSpec((_EB, 4), lambda i: (i, 0)),
        out_shape=jax.ShapeDtypeStruct((E, 4), jnp.float32),
    )(arow, bcol, meta, w2, b2.reshape(1, 64), w3, b3.reshape(1, 4))


def kernel(x, edge_index, edge_attr, batch, params):
    p = params
    at = jnp.clip(x[:, 0].astype(jnp.int32), 0, 10)
    bond = jnp.clip(edge_attr[:, 0].astype(jnp.int32), 0, 4)
    row, col = edge_index[0], edge_index[1]

    atom_table = p['atom_table']
    h1t = jnp.maximum(atom_table @ p['vp_w1'] + p['vp_b1'], 0.0)
    vlt = h1t @ p['vp_w2'] + p['vp_b2']
    pv_tab = jnp.argmax(vlt, axis=-1).astype(jnp.int32) + 1
    h0_tab = jnp.concatenate([
        atom_table,
        jax.nn.one_hot(pv_tab - 1, 8, dtype=jnp.float32),
        jnp.zeros((11, H - 72), jnp.float32),
    ], axis=-1)
    gate_tab = pv_tab.astype(jnp.float32) / 8.0

    wm_h = [p['g%d_wm' % l][:H] for l in range(3)]
    wm_b = [p['g%d_wm' % l][H:] for l in range(3)]
    btm = [p['bond_table'] @ wm_b[l] for l in range(3)]
    hm0_tab = h0_tab @ wm_h[0] + p['g0_bm']

    valence_logits = jnp.take(vlt, at, axis=0)
    pv = jnp.take(pv_tab, at, axis=0)
    pv_f = pv.astype(jnp.float32)
    gate = jnp.take(gate_tab, at, axis=0)[:, None]
    h = jnp.take(h0_tab, at, axis=0)
    hm = jnp.take(hm0_tab, at, axis=0)

    for l in range(3):
        msg = jnp.maximum(
            jnp.take(hm, row, axis=0) + jnp.take(btm[l], bond, axis=0), 0.0)
        agg = jax.ops.segment_sum(msg, col, num_segments=N)
        wu = p['g%d_wu' % l]
        h = _gnn_update(h, agg, gate, wu[:H], wu[H:], p['g%d_bu' % l])
        if l < 2:
            hm = _mm_bias(h, wm_h[l + 1], p['g%d_bm' % (l + 1)])

    degree = jnp.zeros((N,), jnp.float32).at[col].add(1.0)
    w1 = p['btc_w1']
    wv = w1[2 * H:]
    hf, anode, bnode, props, vio = _epilogue(
        h, pv_f[:, None], degree[:, None],
        w1[:H], w1[H:2 * H], wv, p['btc_b1'],
        p['cp_w1'], p['cp_b1'], p['cp_w2'], p['cp_b2'])
    violations = vio[:, 0]

    arow = jnp.take(anode, row, axis=0)
    bcol = jnp.take(bnode, col, axis=0)
    at_f = at.astype(jnp.float32)
    meta = jnp.stack([
        jnp.take(at_f, row), jnp.take(at_f, col),
        jnp.take(pv_f, row), jnp.take(pv_f, col)], axis=1)
    bond_logits = _edge_cls(arow, bcol, meta,
                            p['btc_w2'], p['btc_b2'], p['btc_w3'], p['btc_b3'])

    return (hf, props, valence_logits, bond_logits, violations)


# speedup vs baseline: 1.5346x; 1.4136x over previous
import functools

import jax
import jax.numpy as jnp
from jax import lax
from jax.experimental import pallas as pl
from jax.experimental.pallas import tpu as pltpu
from jax.experimental.pallas import tpu_sc as plsc

N = 10000
NPAD = 10240
E = 160000
H = 256

_NB = 1000
_EB = 1000

_CH = 320
_SUB = 80
_NSUB = _CH // _SUB
_TCH = E // _CH
_EXTRA = _TCH - 16 * (_TCH // 16)
_RPT = NPAD // 16


def _expand_body(at_ref, tab_ref, o_ref):
    oh = jnp.where(
        at_ref[...] == jax.lax.broadcasted_iota(jnp.int32, (_NB, 16), 1),
        1.0, 0.0)
    o_ref[...] = jnp.dot(oh, tab_ref[...], preferred_element_type=jnp.float32,
                         precision=jax.lax.Precision.HIGHEST)


def _expand(at, tab):
    w = tab.shape[1]
    return pl.pallas_call(
        _expand_body,
        grid=(N // _NB,),
        in_specs=[
            pl.BlockSpec((_NB, 1), lambda i: (i, 0)),
            pl.BlockSpec((16, w), lambda i: (0, 0)),
        ],
        out_specs=pl.BlockSpec((_NB, w), lambda i: (i, 0)),
        out_shape=jax.ShapeDtypeStruct((N, w), jnp.float32),
    )(at, tab)


def _mm_bias_body(h_ref, w_ref, b_ref, o_ref):
    o_ref[...] = (
        jnp.dot(h_ref[...], w_ref[...], preferred_element_type=jnp.float32)
        + b_ref[...]
    )


def _mm_bias(h, w, b):
    m, k = h.shape
    n = w.shape[1]
    return pl.pallas_call(
        _mm_bias_body,
        grid=(m // _NB,),
        in_specs=[
            pl.BlockSpec((_NB, k), lambda i: (i, 0)),
            pl.BlockSpec((k, n), lambda i: (0, 0)),
            pl.BlockSpec((1, n), lambda i: (0, 0)),
        ],
        out_specs=pl.BlockSpec((_NB, n), lambda i: (i, 0)),
        out_shape=jax.ShapeDtypeStruct((m, n), jnp.float32),
    )(h, w, b.reshape(1, n))


def _hm5_body(hm_ref, btm_ref, o_ref):
    o_ref[...] = jnp.maximum(hm_ref[...] + btm_ref[0], 0.0)


def _hm5(hm, btm):
    return pl.pallas_call(
        _hm5_body,
        grid=(10, N // _NB),
        in_specs=[
            pl.BlockSpec((_NB, 128), lambda g, i: (i, g % 2)),
            pl.BlockSpec((1, 1, 128), lambda g, i: (g // 2, 0, g % 2)),
        ],
        out_specs=pl.BlockSpec((_NB, 128), lambda g, i: (g * (N // _NB) + i, 0)),
        out_shape=jax.ShapeDtypeStruct((10 * N, 128), jnp.float32),
    )(hm, btm.reshape(5, 1, 256))


def _update_body(h_ref, aa_ref, ab_ref, gate_ref, wa_ref, wb1_ref, wb2_ref,
                 b_ref, o_ref):
    z = (
        jnp.dot(h_ref[...], wa_ref[...], preferred_element_type=jnp.float32)
        + jnp.dot(aa_ref[...], wb1_ref[...], preferred_element_type=jnp.float32)
        + jnp.dot(ab_ref[...], wb2_ref[...], preferred_element_type=jnp.float32)
        + b_ref[...]
    )
    o_ref[...] = jnp.maximum(z, 0.0) * gate_ref[...]


def _gnn_update(h, agg_a, agg_b, gate, wa, wb1, wb2, b):
    return pl.pallas_call(
        _update_body,
        grid=(N // _NB,),
        in_specs=[
            pl.BlockSpec((_NB, H), lambda i: (i, 0)),
            pl.BlockSpec((_NB, 128), lambda i: (i, 0)),
            pl.BlockSpec((_NB, 128), lambda i: (i, 0)),
            pl.BlockSpec((_NB, 1), lambda i: (i, 0)),
            pl.BlockSpec((H, H), lambda i: (0, 0)),
            pl.BlockSpec((128, H), lambda i: (0, 0)),
            pl.BlockSpec((128, H), lambda i: (0, 0)),
            pl.BlockSpec((1, H), lambda i: (0, 0)),
        ],
        out_specs=pl.BlockSpec((_NB, H), lambda i: (i, 0)),
        out_shape=jax.ShapeDtypeStruct((N, H), jnp.float32),
    )(h, agg_a, agg_b, gate, wa, wb1, wb2, b.reshape(1, H))


def _epilogue_body(h_ref, pv_ref, at_ref, deg_ref, wr_ref, wc_ref, wv_ref,
                   b1_ref, cw1_ref, cb1_ref, cw2_ref, cb2_ref,
                   hf_ref, an_ref, bn_ref, pr_ref, vio_ref):
    pv = pv_ref[...]
    vio = jnp.maximum(deg_ref[...] - pv, 0.0)
    vio_ref[...] = vio
    hf = h_ref[...] / (1.0 + vio)
    hf_ref[...] = hf
    wv = wv_ref[...]
    an_ref[...] = (
        jnp.dot(hf, wr_ref[...], preferred_element_type=jnp.float32)
        + pv * wv[0:1, :])
    bn_ref[...] = (
        jnp.dot(hf, wc_ref[...], preferred_element_type=jnp.float32)
        + pv * wv[1:2, :] + b1_ref[...])
    p1 = jnp.maximum(
        jnp.dot(hf, cw1_ref[...], preferred_element_type=jnp.float32)
        + cb1_ref[...], 0.0)
    pr_ref[...] = (
        jnp.dot(p1, cw2_ref[...], preferred_element_type=jnp.float32)
        + cb2_ref[...]
    )


def _epilogue(h, pv_f, at_f, degree, wr, wc, wv, b1, cw1, cb1, cw2, cb2):
    out_shapes = (
        jax.ShapeDtypeStruct((N, H), jnp.float32),
        jax.ShapeDtypeStruct((N, 128), jnp.float32),
        jax.ShapeDtypeStruct((N, 128), jnp.float32),
        jax.ShapeDtypeStruct((N, 32), jnp.float32),
        jax.ShapeDtypeStruct((N, 1), jnp.float32),
    )
    return pl.pallas_call(
        _epilogue_body,
        grid=(N // _NB,),
        in_specs=[
            pl.BlockSpec((_NB, H), lambda i: (i, 0)),
            pl.BlockSpec((_NB, 1), lambda i: (i, 0)),
            pl.BlockSpec((_NB, 1), lambda i: (i, 0)),
            pl.BlockSpec((_NB, 1), lambda i: (i, 0)),
            pl.BlockSpec((H, 128), lambda i: (0, 0)),
            pl.BlockSpec((H, 128), lambda i: (0, 0)),
            pl.BlockSpec((2, 128), lambda i: (0, 0)),
            pl.BlockSpec((1, 128), lambda i: (0, 0)),
            pl.BlockSpec((H, 128), lambda i: (0, 0)),
            pl.BlockSpec((1, 128), lambda i: (0, 0)),
            pl.BlockSpec((128, 32), lambda i: (0, 0)),
            pl.BlockSpec((1, 32), lambda i: (0, 0)),
        ],
        out_specs=[
            pl.BlockSpec((_NB, H), lambda i: (i, 0)),
            pl.BlockSpec((_NB, 128), lambda i: (i, 0)),
            pl.BlockSpec((_NB, 128), lambda i: (i, 0)),
            pl.BlockSpec((_NB, 32), lambda i: (i, 0)),
            pl.BlockSpec((_NB, 1), lambda i: (i, 0)),
        ],
        out_shape=out_shapes,
    )(h, pv_f, at_f, degree, wr, wc, wv, b1.reshape(1, 128),
      cw1, cb1.reshape(1, 128), cw2, cb2.reshape(1, 32))


def _edge_cls_body(f1_ref, at1_ref, at2_ref, pv1_ref, pv2_ref,
                   w2_ref, b2_ref, w3_ref, b3_ref, o_ref):
    f1 = f1_ref[...]
    at1 = at1_ref[...]
    at2 = at2_ref[...]
    pv1 = pv1_ref[...]
    pv2 = pv2_ref[...]
    z = jnp.maximum(
        jnp.dot(f1, w2_ref[...], preferred_element_type=jnp.float32)
        + b2_ref[...], 0.0)
    lg = jnp.dot(z, w3_ref[...], preferred_element_type=jnp.float32) + b3_ref[...]
    noble = (at1 == 4.0) | (at1 == 5.0) | (at2 == 4.0) | (at2 == 5.0)
    m2 = (pv1 <= 2.0) | (pv2 <= 2.0)
    m3 = (pv1 <= 1.0) | (pv2 <= 1.0)
    ci = jax.lax.broadcasted_iota(jnp.int32, lg.shape, 1)
    pen = jnp.where((ci >= 1) & noble, -100.0, 0.0)
    pen = pen + jnp.where((ci == 2) & m2, -50.0, 0.0)
    pen = pen + jnp.where((ci >= 1) & m3, -50.0, 0.0)
    o_ref[...] = lg + pen


def _edge_cls(f1, at1, at2, pv1, pv2, w2, b2, w3, b3):
    return pl.pallas_call(
        _edge_cls_body,
        grid=(E // _EB,),
        in_specs=[
            pl.BlockSpec((_EB, 128), lambda i: (i, 0)),
            pl.BlockSpec((_EB, 1), lambda i: (i, 0)),
            pl.BlockSpec((_EB, 1), lambda i: (i, 0)),
            pl.BlockSpec((_EB, 1), lambda i: (i, 0)),
            pl.BlockSpec((_EB, 1), lambda i: (i, 0)),
            pl.BlockSpec((128, 64), lambda i: (0, 0)),
            pl.BlockSpec((1, 64), lambda i: (0, 0)),
            pl.BlockSpec((64, 4), lambda i: (0, 0)),
            pl.BlockSpec((1, 4), lambda i: (0, 0)),
        ],
        out_specs=pl.BlockSpec((_EB, 4), lambda i: (i, 0)),
        out_shape=jax.ShapeDtypeStruct((E, 4), jnp.float32),
    )(f1, at1.reshape(E, 1), at2.reshape(E, 1), pv1.reshape(E, 1),
      pv2.reshape(E, 1), w2, b2.reshape(1, 64), w3, b3.reshape(1, 4))


def _sc_msg_call(hm5, row, bond, col, zeros2d):
    mesh = plsc.VectorSubcoreMesh(core_axis_name="c", subcore_axis_name="s")
    out_type = jax.ShapeDtypeStruct((2 * N, 128), jnp.float32)

    def body(hm5_h, row_h, bond_h, col_h, z2_h, agg_h,
             rowv, bondv, colv1, keyv, colv, rows, acc, semi, semg, sems):
        c = lax.axis_index("c")
        s = lax.axis_index("s")
        r0_acc = pl.multiple_of(s * _RPT, 8)
        pltpu.sync_copy(z2_h.at[pl.ds(r0_acc, _RPT)],
                        acc.at[pl.ds(r0_acc, _RPT)])
        plsc.subcore_barrier()

        nch = jnp.where(s < _EXTRA, _TCH // 16 + 1, _TCH // 16)

        def chunk(k, carry):
            base = pl.multiple_of((s + 16 * k) * _CH, 8)
            c1 = pltpu.async_copy(row_h.at[pl.ds(base, _CH)], rowv, semi)
            c2 = pltpu.async_copy(bond_h.at[pl.ds(base, _CH)], bondv, semi)
            c3 = pltpu.async_copy(col_h.at[pl.ds(base, _CH)], colv1, semi)
            c1.wait(); c2.wait(); c3.wait()
            cN = c * N
            for j in range(_NSUB):
                for m in range(_SUB // 16):
                    sl = pl.ds(j * _SUB + m * 16, 16)
                    keyv[j, pl.ds(m * 16, 16)] = (
                        bondv[sl] * (2 * N) + rowv[sl] + cN)
                    colv[j, pl.ds(m * 16, 16)] = colv1[sl]
            gs = [pltpu.async_copy(hm5_h.at[keyv.at[j]],
                                   rows.at[pl.ds(j * _SUB, _SUB)], semg)
                  for j in range(_NSUB)]
            for g in gs:
                g.wait()
            ss = [pltpu.async_copy(rows.at[pl.ds(j * _SUB, _SUB)],
                                   acc.at[colv.at[j]], sems, add=True)
                  for j in range(_NSUB)]
            for t in ss:
                t.wait()
            return carry

        lax.fori_loop(0, nch, chunk, 0)
        plsc.subcore_barrier()

        _LAST = N - 15 * _RPT

        @pl.when(s < 15)
        def _():
            pltpu.sync_copy(
                acc.at[pl.ds(r0_acc, _RPT)],
                agg_h.at[pl.ds(pl.multiple_of(c * N + r0_acc, 8), _RPT)])

        @pl.when(s == 15)
        def _():
            pltpu.sync_copy(
                acc.at[pl.ds(15 * _RPT, _LAST)],
                agg_h.at[pl.ds(pl.multiple_of(c * N + 15 * _RPT, 8), _LAST)])

    scratch = [
        pltpu.VMEM((_CH,), jnp.int32),
        pltpu.VMEM((_CH,), jnp.int32),
        pltpu.VMEM((_CH,), jnp.int32),
        pltpu.VMEM((_NSUB, _SUB), jnp.int32),
        pltpu.VMEM((_NSUB, _SUB), jnp.int32),
        pltpu.VMEM((_CH, 128), jnp.float32),
        pltpu.VMEM_SHARED((NPAD, 128), jnp.float32),
        pltpu.SemaphoreType.DMA,
        pltpu.SemaphoreType.DMA,
        pltpu.SemaphoreType.DMA,
    ]
    fn = pl.kernel(body, out_type=out_type, mesh=mesh, scratch_types=scratch)
    return fn(hm5, row, bond, col, zeros2d)


_CCH = 320
_CNSUB = _CCH // _SUB
_CTCH = E // _CCH
_CEXTRA = _CTCH - 32 * (_CTCH // 32)


def _sc_edge_gather_call(an, bn, row, col):
    mesh = plsc.VectorSubcoreMesh(core_axis_name="c", subcore_axis_name="s")
    out_type = jax.ShapeDtypeStruct((E, 128), jnp.float32)

    def body(an_h, bn_h, row_h, col_h, f1_h,
             rowv, colv1, ridx, cidx, bufa, bufb, semi, semg, semo):
        c = lax.axis_index("c")
        s = lax.axis_index("s")
        wid = c * 16 + s
        nch = jnp.where(wid < _CEXTRA, _CTCH // 32 + 1, _CTCH // 32)

        def chunk(k, carry):
            base = pl.multiple_of((wid + 32 * k) * _CCH, 8)
            c1 = pltpu.async_copy(row_h.at[pl.ds(base, _CCH)], rowv, semi)
            c2 = pltpu.async_copy(col_h.at[pl.ds(base, _CCH)], colv1, semi)
            c1.wait(); c2.wait()
            for j in range(_CNSUB):
                for m in range(_SUB // 16):
                    sl = pl.ds(j * _SUB + m * 16, 16)
                    ridx[j, pl.ds(m * 16, 16)] = rowv[sl]
                    cidx[j, pl.ds(m * 16, 16)] = colv1[sl]
            gs = []
            for j in range(_CNSUB):
                gs.append(pltpu.async_copy(
                    an_h.at[ridx.at[j]],
                    bufa.at[pl.ds(j * _SUB, _SUB)], semg))
                gs.append(pltpu.async_copy(
                    bn_h.at[cidx.at[j]],
                    bufb.at[pl.ds(j * _SUB, _SUB)], semg))
            for g in gs:
                g.wait()

            def edge(e, cc):
                for g in range(8):
                    sl = pl.ds(g * 16, 16)
                    bufa[e, sl] = jnp.maximum(bufa[e, sl] + bufb[e, sl], 0.0)
                return cc

            lax.fori_loop(0, _CCH, edge, 0)
            pltpu.async_copy(bufa, f1_h.at[pl.ds(base, _CCH)], semo).wait()
            return carry

        lax.fori_loop(0, nch, chunk, 0)

    scratch = [
        pltpu.VMEM((_CCH,), jnp.int32),
        pltpu.VMEM((_CCH,), jnp.int32),
        pltpu.VMEM((_CNSUB, _SUB), jnp.int32),
        pltpu.VMEM((_CNSUB, _SUB), jnp.int32),
        pltpu.VMEM((_CCH, 128), jnp.float32),
        pltpu.VMEM((_CCH, 128), jnp.float32),
        pltpu.SemaphoreType.DMA,
        pltpu.SemaphoreType.DMA,
        pltpu.SemaphoreType.DMA,
    ]
    fn = pl.kernel(body, out_type=out_type, mesh=mesh, scratch_types=scratch)
    return fn(an, bn, row, col)


def kernel(x, edge_index, edge_attr, batch, params):
    p = params
    at = jnp.clip(x[:, 0].astype(jnp.int32), 0, 10)
    bond = jnp.clip(edge_attr[:, 0].astype(jnp.int32), 0, 4)
    row = edge_index[0].astype(jnp.int32)
    col = edge_index[1].astype(jnp.int32)

    atom_table = p['atom_table']
    h1t = jnp.maximum(atom_table @ p['vp_w1'] + p['vp_b1'], 0.0)
    vlt = h1t @ p['vp_w2'] + p['vp_b2']
    pv_tab = jnp.argmax(vlt, axis=-1).astype(jnp.int32) + 1
    h0_tab = jnp.concatenate([
        atom_table,
        jax.nn.one_hot(pv_tab - 1, 8, dtype=jnp.float32),
        jnp.zeros((11, H - 72), jnp.float32),
    ], axis=-1)

    wm_h = [p['g%d_wm' % l][:H] for l in range(3)]
    wm_b = [p['g%d_wm' % l][H:] for l in range(3)]
    btm = [p['bond_table'] @ wm_b[l] for l in range(3)]
    hm0_tab = h0_tab @ wm_h[0] + p['g0_bm']

    pv_tab_f = pv_tab.astype(jnp.float32)[:, None]
    big = jnp.concatenate([
        h0_tab, hm0_tab, vlt, pv_tab_f, pv_tab_f / 8.0,
        jnp.zeros((11, 640 - 522), jnp.float32)], axis=1)
    big = jnp.concatenate([big, jnp.zeros((5, 640), jnp.float32)], axis=0)

    nodevals = _expand(at[:, None], big)
    h = nodevals[:, :256]
    hm = nodevals[:, 256:512]
    valence_logits = nodevals[:, 512:520]
    pv_f = nodevals[:, 520:521]
    gate = nodevals[:, 521:522]

    zeros2d = jnp.zeros((NPAD, 128), jnp.float32)

    degree = jnp.zeros((N,), jnp.float32).at[col].add(1.0)
    for l in range(3):
        hm5 = _hm5(hm, btm[l])
        agg2 = _sc_msg_call(hm5, row, bond, col, zeros2d)
        wu = p['g%d_wu' % l]
        h = _gnn_update(h, agg2[:N], agg2[N:], gate,
                        wu[:H], wu[H:H + 128], wu[H + 128:],
                        p['g%d_bu' % l])
        if l < 2:
            hm = _mm_bias(h, wm_h[l + 1], p['g%d_bm' % (l + 1)])

    w1 = p['btc_w1']
    at_f = at.astype(jnp.float32)
    hf, anode, bnode, props, vio = _epilogue(
        h, pv_f, at_f[:, None], degree[:, None],
        w1[:H], w1[H:2 * H], w1[2 * H:], p['btc_b1'],
        p['cp_w1'], p['cp_b1'], p['cp_w2'], p['cp_b2'])
    violations = vio[:, 0]

    f1 = _sc_edge_gather_call(anode, bnode, row, col)
    pv1d = pv_f[:, 0]
    at1 = jnp.take(at_f, row)
    at2 = jnp.take(at_f, col)
    pv1 = jnp.take(pv1d, row)
    pv2 = jnp.take(pv1d, col)
    bond_logits = _edge_cls(f1, at1, at2, pv1, pv2,
                            p['btc_w2'], p['btc_b2'],
                            p['btc_w3'], p['btc_b3'])

    return (hf, props, valence_logits, bond_logits, violations)


# speedup vs baseline: 5.2041x; 3.3912x over previous
import functools

import jax
import jax.numpy as jnp
from jax import lax
from jax.experimental import pallas as pl
from jax.experimental.pallas import tpu as pltpu
from jax.experimental.pallas import tpu_sc as plsc

N = 10000
NPAD = 10240
E = 160000
H = 256

_NB = 1000
_EB = 1000

_CH = 320
_SUB = 80
_NSUB = _CH // _SUB
_TCH = E // _CH
_EXTRA = _TCH - 16 * (_TCH // 16)
_RPT = NPAD // 16


def _expand_body(at_ref, tab_ref, o_ref):
    oh = jnp.where(
        at_ref[...] == jax.lax.broadcasted_iota(jnp.int32, (_NB, 16), 1),
        1.0, 0.0)
    o_ref[...] = jnp.dot(oh, tab_ref[...], preferred_element_type=jnp.float32,
                         precision=jax.lax.Precision.HIGHEST)


def _expand(at, tab):
    w = tab.shape[1]
    return pl.pallas_call(
        _expand_body,
        grid=(N // _NB,),
        in_specs=[
            pl.BlockSpec((_NB, 1), lambda i: (i, 0)),
            pl.BlockSpec((16, w), lambda i: (0, 0)),
        ],
        out_specs=pl.BlockSpec((_NB, w), lambda i: (i, 0)),
        out_shape=jax.ShapeDtypeStruct((N, w), jnp.float32),
    )(at, tab)


def _mm_bias_body(h_ref, w_ref, b_ref, o_ref):
    o_ref[...] = (
        jnp.dot(h_ref[...], w_ref[...], preferred_element_type=jnp.float32)
        + b_ref[...]
    )


def _mm_bias(h, w, b):
    m, k = h.shape
    n = w.shape[1]
    return pl.pallas_call(
        _mm_bias_body,
        grid=(m // _NB,),
        in_specs=[
            pl.BlockSpec((_NB, k), lambda i: (i, 0)),
            pl.BlockSpec((k, n), lambda i: (0, 0)),
            pl.BlockSpec((1, n), lambda i: (0, 0)),
        ],
        out_specs=pl.BlockSpec((_NB, n), lambda i: (i, 0)),
        out_shape=jax.ShapeDtypeStruct((m, n), jnp.float32),
    )(h, w, b.reshape(1, n))


def _hm5_body(hm_ref, btm_ref, o_ref):
    o_ref[...] = jnp.maximum(hm_ref[...] + btm_ref[0], 0.0)


def _hm5(hm, btm):
    return pl.pallas_call(
        _hm5_body,
        grid=(10, N // _NB),
        in_specs=[
            pl.BlockSpec((_NB, 128), lambda g, i: (i, g % 2)),
            pl.BlockSpec((1, 1, 128), lambda g, i: (g // 2, 0, g % 2)),
        ],
        out_specs=pl.BlockSpec((_NB, 128), lambda g, i: (g * (N // _NB) + i, 0)),
        out_shape=jax.ShapeDtypeStruct((10 * N, 128), jnp.float32),
    )(hm, btm.reshape(5, 1, 256))


def _update_body(h_ref, aa_ref, ab_ref, gate_ref, wa_ref, wb1_ref, wb2_ref,
                 b_ref, o_ref):
    z = (
        jnp.dot(h_ref[...], wa_ref[...], preferred_element_type=jnp.float32)
        + jnp.dot(aa_ref[...], wb1_ref[...], preferred_element_type=jnp.float32)
        + jnp.dot(ab_ref[...], wb2_ref[...], preferred_element_type=jnp.float32)
        + b_ref[...]
    )
    o_ref[...] = jnp.maximum(z, 0.0) * gate_ref[...]


def _gnn_update(h, agg_a, agg_b, gate, wa, wb1, wb2, b):
    return pl.pallas_call(
        _update_body,
        grid=(N // _NB,),
        in_specs=[
            pl.BlockSpec((_NB, H), lambda i: (i, 0)),
            pl.BlockSpec((_NB, 128), lambda i: (i, 0)),
            pl.BlockSpec((_NB, 128), lambda i: (i, 0)),
            pl.BlockSpec((_NB, 1), lambda i: (i, 0)),
            pl.BlockSpec((H, H), lambda i: (0, 0)),
            pl.BlockSpec((128, H), lambda i: (0, 0)),
            pl.BlockSpec((128, H), lambda i: (0, 0)),
            pl.BlockSpec((1, H), lambda i: (0, 0)),
        ],
        out_specs=pl.BlockSpec((_NB, H), lambda i: (i, 0)),
        out_shape=jax.ShapeDtypeStruct((N, H), jnp.float32),
    )(h, agg_a, agg_b, gate, wa, wb1, wb2, b.reshape(1, H))


def _epilogue_body(h_ref, pv_ref, at_ref, deg_ref, wr_ref, wc_ref, wv_ref,
                   b1_ref, cw1_ref, cb1_ref, cw2_ref, cb2_ref,
                   hf_ref, an_ref, bn_ref, pr_ref, vio_ref):
    pv = pv_ref[...]
    vio = jnp.maximum(deg_ref[...] - pv, 0.0)
    vio_ref[...] = vio
    hf = h_ref[...] / (1.0 + vio)
    hf_ref[...] = hf
    wv = wv_ref[...]
    ci = jax.lax.broadcasted_iota(jnp.int32, (_NB, 128), 1)
    tail = (jnp.where(ci == 0, at_ref[...], 0.0)
            + jnp.where(ci == 1, pv, 0.0))
    an = (jnp.dot(hf, wr_ref[...], preferred_element_type=jnp.float32)
          + pv * wv[0:1, :])
    bn = (jnp.dot(hf, wc_ref[...], preferred_element_type=jnp.float32)
          + pv * wv[1:2, :] + b1_ref[...])
    an_ref[...] = jnp.concatenate([an, tail], axis=1)
    bn_ref[...] = jnp.concatenate([bn, tail], axis=1)
    p1 = jnp.maximum(
        jnp.dot(hf, cw1_ref[...], preferred_element_type=jnp.float32)
        + cb1_ref[...], 0.0)
    pr_ref[...] = (
        jnp.dot(p1, cw2_ref[...], preferred_element_type=jnp.float32)
        + cb2_ref[...]
    )


def _epilogue(h, pv_f, at_f, degree, wr, wc, wv, b1, cw1, cb1, cw2, cb2):
    out_shapes = (
        jax.ShapeDtypeStruct((N, H), jnp.float32),
        jax.ShapeDtypeStruct((N, 256), jnp.float32),
        jax.ShapeDtypeStruct((N, 256), jnp.float32),
        jax.ShapeDtypeStruct((N, 32), jnp.float32),
        jax.ShapeDtypeStruct((N, 1), jnp.float32),
    )
    return pl.pallas_call(
        _epilogue_body,
        grid=(N // _NB,),
        in_specs=[
            pl.BlockSpec((_NB, H), lambda i: (i, 0)),
            pl.BlockSpec((_NB, 1), lambda i: (i, 0)),
            pl.BlockSpec((_NB, 1), lambda i: (i, 0)),
            pl.BlockSpec((_NB, 1), lambda i: (i, 0)),
            pl.BlockSpec((H, 128), lambda i: (0, 0)),
            pl.BlockSpec((H, 128), lambda i: (0, 0)),
            pl.BlockSpec((2, 128), lambda i: (0, 0)),
            pl.BlockSpec((1, 128), lambda i: (0, 0)),
            pl.BlockSpec((H, 128), lambda i: (0, 0)),
            pl.BlockSpec((1, 128), lambda i: (0, 0)),
            pl.BlockSpec((128, 32), lambda i: (0, 0)),
            pl.BlockSpec((1, 32), lambda i: (0, 0)),
        ],
        out_specs=[
            pl.BlockSpec((_NB, H), lambda i: (i, 0)),
            pl.BlockSpec((_NB, 256), lambda i: (i, 0)),
            pl.BlockSpec((_NB, 256), lambda i: (i, 0)),
            pl.BlockSpec((_NB, 32), lambda i: (i, 0)),
            pl.BlockSpec((_NB, 1), lambda i: (i, 0)),
        ],
        out_shape=out_shapes,
    )(h, pv_f, at_f, degree, wr, wc, wv, b1.reshape(1, 128),
      cw1, cb1.reshape(1, 128), cw2, cb2.reshape(1, 32))


def _edge_cls_body(f1_ref, mt_ref,
                   w2_ref, b2_ref, w3_ref, b3_ref, o_ref):
    f1 = f1_ref[...]
    mt = mt_ref[...]
    at1 = mt[:, 0:1]
    pv1 = mt[:, 1:2]
    at2 = mt[:, 16:17]
    pv2 = mt[:, 17:18]
    z = jnp.maximum(
        jnp.dot(f1, w2_ref[...], preferred_element_type=jnp.float32)
        + b2_ref[...], 0.0)
    lg = jnp.dot(z, w3_ref[...], preferred_element_type=jnp.float32) + b3_ref[...]
    noble = (at1 == 4.0) | (at1 == 5.0) | (at2 == 4.0) | (at2 == 5.0)
    m2 = (pv1 <= 2.0) | (pv2 <= 2.0)
    m3 = (pv1 <= 1.0) | (pv2 <= 1.0)
    ci = jax.lax.broadcasted_iota(jnp.int32, lg.shape, 1)
    pen = jnp.where((ci >= 1) & noble, -100.0, 0.0)
    pen = pen + jnp.where((ci == 2) & m2, -50.0, 0.0)
    pen = pen + jnp.where((ci >= 1) & m3, -50.0, 0.0)
    o_ref[...] = lg + pen


def _edge_cls(f1, mt, w2, b2, w3, b3):
    return pl.pallas_call(
        _edge_cls_body,
        grid=(E // _EB,),
        in_specs=[
            pl.BlockSpec((_EB, 128), lambda i: (i, 0)),
            pl.BlockSpec((_EB, 32), lambda i: (i, 0)),
            pl.BlockSpec((128, 64), lambda i: (0, 0)),
            pl.BlockSpec((1, 64), lambda i: (0, 0)),
            pl.BlockSpec((64, 4), lambda i: (0, 0)),
            pl.BlockSpec((1, 4), lambda i: (0, 0)),
        ],
        out_specs=pl.BlockSpec((_EB, 4), lambda i: (i, 0)),
        out_shape=jax.ShapeDtypeStruct((E, 4), jnp.float32),
    )(f1, mt, w2, b2.reshape(1, 64), w3, b3.reshape(1, 4))


def _sc_msg_call(hm5, row, bond, col, zeros2d):
    mesh = plsc.VectorSubcoreMesh(core_axis_name="c", subcore_axis_name="s")
    out_type = jax.ShapeDtypeStruct((2 * N, 128), jnp.float32)

    def body(hm5_h, row_h, bond_h, col_h, z2_h, agg_h,
             rowv, bondv, colv1, keyv, colv, rows, acc, semi, semg, sems):
        c = lax.axis_index("c")
        s = lax.axis_index("s")
        r0_acc = pl.multiple_of(s * _RPT, 8)
        pltpu.sync_copy(z2_h.at[pl.ds(r0_acc, _RPT)],
                        acc.at[pl.ds(r0_acc, _RPT)])
        plsc.subcore_barrier()

        nch = jnp.where(s < _EXTRA, _TCH // 16 + 1, _TCH // 16)

        def chunk(k, carry):
            base = pl.multiple_of((s + 16 * k) * _CH, 8)
            c1 = pltpu.async_copy(row_h.at[pl.ds(base, _CH)], rowv, semi)
            c2 = pltpu.async_copy(bond_h.at[pl.ds(base, _CH)], bondv, semi)
            c3 = pltpu.async_copy(col_h.at[pl.ds(base, _CH)], colv1, semi)
            c1.wait(); c2.wait(); c3.wait()
            cN = c * N
            for j in range(_NSUB):
                for m in range(_SUB // 16):
                    sl = pl.ds(j * _SUB + m * 16, 16)
                    keyv[j, pl.ds(m * 16, 16)] = (
                        bondv[sl] * (2 * N) + rowv[sl] + cN)
                    colv[j, pl.ds(m * 16, 16)] = colv1[sl]
            gs = [pltpu.async_copy(hm5_h.at[keyv.at[j]],
                                   rows.at[pl.ds(j * _SUB, _SUB)], semg)
                  for j in range(_NSUB)]
            for g in gs:
                g.wait()
            ss = [pltpu.async_copy(rows.at[pl.ds(j * _SUB, _SUB)],
                                   acc.at[colv.at[j]], sems, add=True)
                  for j in range(_NSUB)]
            for t in ss:
                t.wait()
            return carry

        lax.fori_loop(0, nch, chunk, 0)
        plsc.subcore_barrier()

        _LAST = N - 15 * _RPT

        @pl.when(s < 15)
        def _():
            pltpu.sync_copy(
                acc.at[pl.ds(r0_acc, _RPT)],
                agg_h.at[pl.ds(pl.multiple_of(c * N + r0_acc, 8), _RPT)])

        @pl.when(s == 15)
        def _():
            pltpu.sync_copy(
                acc.at[pl.ds(15 * _RPT, _LAST)],
                agg_h.at[pl.ds(pl.multiple_of(c * N + 15 * _RPT, 8), _LAST)])

    scratch = [
        pltpu.VMEM((_CH,), jnp.int32),
        pltpu.VMEM((_CH,), jnp.int32),
        pltpu.VMEM((_CH,), jnp.int32),
        pltpu.VMEM((_NSUB, _SUB), jnp.int32),
        pltpu.VMEM((_NSUB, _SUB), jnp.int32),
        pltpu.VMEM((_CH, 128), jnp.float32),
        pltpu.VMEM_SHARED((NPAD, 128), jnp.float32),
        pltpu.SemaphoreType.DMA,
        pltpu.SemaphoreType.DMA,
        pltpu.SemaphoreType.DMA,
    ]
    fn = pl.kernel(body, out_type=out_type, mesh=mesh, scratch_types=scratch)
    return fn(hm5, row, bond, col, zeros2d)


_CCH = 80
_CTCH = E // _CCH
_CEXTRA = _CTCH - 32 * (_CTCH // 32)


def _sc_edge_gather_call(an, bn, row, col):
    mesh = plsc.VectorSubcoreMesh(core_axis_name="c", subcore_axis_name="s")
    out_type = (jax.ShapeDtypeStruct((E, 128), jnp.float32),
                jax.ShapeDtypeStruct((E, 32), jnp.float32))

    def body(an_h, bn_h, row_h, col_h, f1_h, mt_h,
             ridx, cidx, bufa, bufb, obuf, mbuf, semi, semg, semo):
        c = lax.axis_index("c")
        s = lax.axis_index("s")
        wid = c * 16 + s
        nch = jnp.where(wid < _CEXTRA, _CTCH // 32 + 1, _CTCH // 32)

        def chunk(k, carry):
            base = pl.multiple_of((wid + 32 * k) * _CCH, 8)
            c1 = pltpu.async_copy(row_h.at[pl.ds(base, _CCH)], ridx, semi)
            c2 = pltpu.async_copy(col_h.at[pl.ds(base, _CCH)], cidx, semi)
            c1.wait(); c2.wait()
            g1 = pltpu.async_copy(an_h.at[ridx], bufa, semg)
            g2 = pltpu.async_copy(bn_h.at[cidx], bufb, semg)
            g1.wait(); g2.wait()

            def edge(e, cc):
                for g in range(8):
                    sl = pl.ds(g * 16, 16)
                    obuf[e, sl] = jnp.maximum(bufa[e, sl] + bufb[e, sl], 0.0)
                mbuf[e, pl.ds(0, 16)] = bufa[e, pl.ds(128, 16)]
                mbuf[e, pl.ds(16, 16)] = bufb[e, pl.ds(128, 16)]
                return cc

            lax.fori_loop(0, _CCH, edge, 0)
            o1 = pltpu.async_copy(obuf, f1_h.at[pl.ds(base, _CCH)], semo)
            o2 = pltpu.async_copy(mbuf, mt_h.at[pl.ds(base, _CCH)], semo)
            o1.wait(); o2.wait()
            return carry

        lax.fori_loop(0, nch, chunk, 0)

    scratch = [
        pltpu.VMEM((_CCH,), jnp.int32),
        pltpu.VMEM((_CCH,), jnp.int32),
        pltpu.VMEM((_CCH, 256), jnp.float32),
        pltpu.VMEM((_CCH, 256), jnp.float32),
        pltpu.VMEM((_CCH, 128), jnp.float32),
        pltpu.VMEM((_CCH, 32), jnp.float32),
        pltpu.SemaphoreType.DMA,
        pltpu.SemaphoreType.DMA,
        pltpu.SemaphoreType.DMA,
    ]
    fn = pl.kernel(body, out_type=out_type, mesh=mesh, scratch_types=scratch)
    return fn(an, bn, row, col)


def kernel(x, edge_index, edge_attr, batch, params):
    p = params
    at = jnp.clip(x[:, 0].astype(jnp.int32), 0, 10)
    bond = jnp.clip(edge_attr[:, 0].astype(jnp.int32), 0, 4)
    row = edge_index[0].astype(jnp.int32)
    col = edge_index[1].astype(jnp.int32)

    atom_table = p['atom_table']
    h1t = jnp.maximum(atom_table @ p['vp_w1'] + p['vp_b1'], 0.0)
    vlt = h1t @ p['vp_w2'] + p['vp_b2']
    pv_tab = jnp.argmax(vlt, axis=-1).astype(jnp.int32) + 1
    h0_tab = jnp.concatenate([
        atom_table,
        jax.nn.one_hot(pv_tab - 1, 8, dtype=jnp.float32),
        jnp.zeros((11, H - 72), jnp.float32),
    ], axis=-1)

    wm_h = [p['g%d_wm' % l][:H] for l in range(3)]
    wm_b = [p['g%d_wm' % l][H:] for l in range(3)]
    btm = [p['bond_table'] @ wm_b[l] for l in range(3)]
    hm0_tab = h0_tab @ wm_h[0] + p['g0_bm']

    pv_tab_f = pv_tab.astype(jnp.float32)[:, None]
    big = jnp.concatenate([
        h0_tab, hm0_tab, vlt, pv_tab_f, pv_tab_f / 8.0,
        jnp.zeros((11, 640 - 522), jnp.float32)], axis=1)
    big = jnp.concatenate([big, jnp.zeros((5, 640), jnp.float32)], axis=0)

    nodevals = _expand(at[:, None], big)
    h = nodevals[:, :256]
    hm = nodevals[:, 256:512]
    valence_logits = nodevals[:, 512:520]
    pv_f = nodevals[:, 520:521]
    gate = nodevals[:, 521:522]

    zeros2d = jnp.zeros((NPAD, 128), jnp.float32)

    degree = jnp.zeros((N,), jnp.float32).at[col].add(1.0)
    for l in range(3):
        hm5 = _hm5(hm, btm[l])
        agg2 = _sc_msg_call(hm5, row, bond, col, zeros2d)
        wu = p['g%d_wu' % l]
        h = _gnn_update(h, agg2[:N], agg2[N:], gate,
                        wu[:H], wu[H:H + 128], wu[H + 128:],
                        p['g%d_bu' % l])
        if l < 2:
            hm = _mm_bias(h, wm_h[l + 1], p['g%d_bm' % (l + 1)])

    w1 = p['btc_w1']
    at_f = at.astype(jnp.float32)
    hf, anode, bnode, props, vio = _epilogue(
        h, pv_f, at_f[:, None], degree[:, None],
        w1[:H], w1[H:2 * H], w1[2 * H:], p['btc_b1'],
        p['cp_w1'], p['cp_b1'], p['cp_w2'], p['cp_b2'])
    violations = vio[:, 0]

    f1, meta = _sc_edge_gather_call(anode, bnode, row, col)
    bond_logits = _edge_cls(f1, meta,
                            p['btc_w2'], p['btc_b2'],
                            p['btc_w3'], p['btc_b3'])

    return (hf, props, valence_logits, bond_logits, violations)


# speedup vs baseline: 5.2869x; 1.0159x over previous
import functools

import jax
import jax.numpy as jnp
from jax import lax
from jax.experimental import pallas as pl
from jax.experimental.pallas import tpu as pltpu
from jax.experimental.pallas import tpu_sc as plsc

N = 10000
NPAD = 10240
E = 160000
H = 256

_NB = 1000
_EB = 1000

_CH = 320
_SUB = 80
_NSUB = _CH // _SUB
_TCH = E // _CH
_EXTRA = _TCH - 16 * (_TCH // 16)
_RPT = NPAD // 16


def _expand_body(at_ref, tab_ref, o_ref):
    oh = jnp.where(
        at_ref[...] == jax.lax.broadcasted_iota(jnp.int32, (_NB, 16), 1),
        1.0, 0.0)
    o_ref[...] = jnp.dot(oh, tab_ref[...], preferred_element_type=jnp.float32,
                         precision=jax.lax.Precision.HIGHEST)


def _expand(at, tab):
    w = tab.shape[1]
    return pl.pallas_call(
        _expand_body,
        grid=(N // _NB,),
        in_specs=[
            pl.BlockSpec((_NB, 1), lambda i: (i, 0)),
            pl.BlockSpec((16, w), lambda i: (0, 0)),
        ],
        out_specs=pl.BlockSpec((_NB, w), lambda i: (i, 0)),
        out_shape=jax.ShapeDtypeStruct((N, w), jnp.float32),
    )(at, tab)


def _mm_bias_body(h_ref, w_ref, b_ref, o_ref):
    o_ref[...] = (
        jnp.dot(h_ref[...], w_ref[...], preferred_element_type=jnp.float32)
        + b_ref[...]
    )


def _mm_bias(h, w, b):
    m, k = h.shape
    n = w.shape[1]
    return pl.pallas_call(
        _mm_bias_body,
        grid=(m // _NB,),
        in_specs=[
            pl.BlockSpec((_NB, k), lambda i: (i, 0)),
            pl.BlockSpec((k, n), lambda i: (0, 0)),
            pl.BlockSpec((1, n), lambda i: (0, 0)),
        ],
        out_specs=pl.BlockSpec((_NB, n), lambda i: (i, 0)),
        out_shape=jax.ShapeDtypeStruct((m, n), jnp.float32),
    )(h, w, b.reshape(1, n))


def _hm5_body(hm_ref, btm_ref, o_ref):
    o_ref[...] = jnp.maximum(hm_ref[...] + btm_ref[0], 0.0)


def _hm5(hm, btm):
    return pl.pallas_call(
        _hm5_body,
        grid=(10, N // _NB),
        in_specs=[
            pl.BlockSpec((_NB, 128), lambda g, i: (i, g % 2)),
            pl.BlockSpec((1, 1, 128), lambda g, i: (g // 2, 0, g % 2)),
        ],
        out_specs=pl.BlockSpec((_NB, 128), lambda g, i: (g * (N // _NB) + i, 0)),
        out_shape=jax.ShapeDtypeStruct((10 * N, 128), jnp.float32),
    )(hm, btm.reshape(5, 1, 256))


def _update_body(h_ref, aa_ref, ab_ref, gate_ref, wa_ref, wb1_ref, wb2_ref,
                 b_ref, o_ref):
    z = (
        jnp.dot(h_ref[...], wa_ref[...], preferred_element_type=jnp.float32)
        + jnp.dot(aa_ref[...], wb1_ref[...], preferred_element_type=jnp.float32)
        + jnp.dot(ab_ref[...], wb2_ref[...], preferred_element_type=jnp.float32)
        + b_ref[...]
    )
    o_ref[...] = jnp.maximum(z, 0.0) * gate_ref[...]


def _gnn_update(h, agg_a, agg_b, gate, wa, wb1, wb2, b):
    return pl.pallas_call(
        _update_body,
        grid=(N // _NB,),
        in_specs=[
            pl.BlockSpec((_NB, H), lambda i: (i, 0)),
            pl.BlockSpec((_NB, 128), lambda i: (i, 0)),
            pl.BlockSpec((_NB, 128), lambda i: (i, 0)),
            pl.BlockSpec((_NB, 1), lambda i: (i, 0)),
            pl.BlockSpec((H, H), lambda i: (0, 0)),
            pl.BlockSpec((128, H), lambda i: (0, 0)),
            pl.BlockSpec((128, H), lambda i: (0, 0)),
            pl.BlockSpec((1, H), lambda i: (0, 0)),
        ],
        out_specs=pl.BlockSpec((_NB, H), lambda i: (i, 0)),
        out_shape=jax.ShapeDtypeStruct((N, H), jnp.float32),
    )(h, agg_a, agg_b, gate, wa, wb1, wb2, b.reshape(1, H))


def _epilogue_body(h_ref, pv_ref, at_ref, deg_ref, wr_ref, wc_ref, wv_ref,
                   b1_ref, cw1_ref, cb1_ref, cw2_ref, cb2_ref,
                   hf_ref, an_ref, bn_ref, pr_ref, vio_ref):
    pv = pv_ref[...]
    vio = jnp.maximum(deg_ref[...] - pv, 0.0)
    vio_ref[...] = vio
    hf = h_ref[...] / (1.0 + vio)
    hf_ref[...] = hf
    wv = wv_ref[...]
    ci = jax.lax.broadcasted_iota(jnp.int32, (_NB, 128), 1)
    tail = (jnp.where(ci == 0, at_ref[...], 0.0)
            + jnp.where(ci == 1, pv, 0.0))
    an = (jnp.dot(hf, wr_ref[...], preferred_element_type=jnp.float32)
          + pv * wv[0:1, :])
    bn = (jnp.dot(hf, wc_ref[...], preferred_element_type=jnp.float32)
          + pv * wv[1:2, :] + b1_ref[...])
    an_ref[...] = jnp.concatenate([an, tail], axis=1)
    bn_ref[...] = jnp.concatenate([bn, tail], axis=1)
    p1 = jnp.maximum(
        jnp.dot(hf, cw1_ref[...], preferred_element_type=jnp.float32)
        + cb1_ref[...], 0.0)
    pr_ref[...] = (
        jnp.dot(p1, cw2_ref[...], preferred_element_type=jnp.float32)
        + cb2_ref[...]
    )


def _epilogue(h, pv_f, at_f, degree, wr, wc, wv, b1, cw1, cb1, cw2, cb2):
    out_shapes = (
        jax.ShapeDtypeStruct((N, H), jnp.float32),
        jax.ShapeDtypeStruct((N, 256), jnp.float32),
        jax.ShapeDtypeStruct((N, 256), jnp.float32),
        jax.ShapeDtypeStruct((N, 32), jnp.float32),
        jax.ShapeDtypeStruct((N, 1), jnp.float32),
    )
    return pl.pallas_call(
        _epilogue_body,
        grid=(N // _NB,),
        in_specs=[
            pl.BlockSpec((_NB, H), lambda i: (i, 0)),
            pl.BlockSpec((_NB, 1), lambda i: (i, 0)),
            pl.BlockSpec((_NB, 1), lambda i: (i, 0)),
            pl.BlockSpec((_NB, 1), lambda i: (i, 0)),
            pl.BlockSpec((H, 128), lambda i: (0, 0)),
            pl.BlockSpec((H, 128), lambda i: (0, 0)),
            pl.BlockSpec((2, 128), lambda i: (0, 0)),
            pl.BlockSpec((1, 128), lambda i: (0, 0)),
            pl.BlockSpec((H, 128), lambda i: (0, 0)),
            pl.BlockSpec((1, 128), lambda i: (0, 0)),
            pl.BlockSpec((128, 32), lambda i: (0, 0)),
            pl.BlockSpec((1, 32), lambda i: (0, 0)),
        ],
        out_specs=[
            pl.BlockSpec((_NB, H), lambda i: (i, 0)),
            pl.BlockSpec((_NB, 256), lambda i: (i, 0)),
            pl.BlockSpec((_NB, 256), lambda i: (i, 0)),
            pl.BlockSpec((_NB, 32), lambda i: (i, 0)),
            pl.BlockSpec((_NB, 1), lambda i: (i, 0)),
        ],
        out_shape=out_shapes,
    )(h, pv_f, at_f, degree, wr, wc, wv, b1.reshape(1, 128),
      cw1, cb1.reshape(1, 128), cw2, cb2.reshape(1, 32))


def _edge_cls_body(f1_ref, mt_ref,
                   w2_ref, b2_ref, w3_ref, b3_ref, o_ref):
    f1 = f1_ref[...]
    mt = mt_ref[...]
    at1 = mt[:, 0:1]
    pv1 = mt[:, 1:2]
    at2 = mt[:, 16:17]
    pv2 = mt[:, 17:18]
    z = jnp.maximum(
        jnp.dot(f1, w2_ref[...], preferred_element_type=jnp.float32)
        + b2_ref[...], 0.0)
    lg = jnp.dot(z, w3_ref[...], preferred_element_type=jnp.float32) + b3_ref[...]
    noble = (at1 == 4.0) | (at1 == 5.0) | (at2 == 4.0) | (at2 == 5.0)
    m2 = (pv1 <= 2.0) | (pv2 <= 2.0)
    m3 = (pv1 <= 1.0) | (pv2 <= 1.0)
    ci = jax.lax.broadcasted_iota(jnp.int32, lg.shape, 1)
    pen = jnp.where((ci >= 1) & noble, -100.0, 0.0)
    pen = pen + jnp.where((ci == 2) & m2, -50.0, 0.0)
    pen = pen + jnp.where((ci >= 1) & m3, -50.0, 0.0)
    o_ref[...] = lg + pen


def _edge_cls(f1, mt, w2, b2, w3, b3):
    return pl.pallas_call(
        _edge_cls_body,
        grid=(E // _EB,),
        in_specs=[
            pl.BlockSpec((_EB, 128), lambda i: (i, 0)),
            pl.BlockSpec((_EB, 32), lambda i: (i, 0)),
            pl.BlockSpec((128, 64), lambda i: (0, 0)),
            pl.BlockSpec((1, 64), lambda i: (0, 0)),
            pl.BlockSpec((64, 4), lambda i: (0, 0)),
            pl.BlockSpec((1, 4), lambda i: (0, 0)),
        ],
        out_specs=pl.BlockSpec((_EB, 4), lambda i: (i, 0)),
        out_shape=jax.ShapeDtypeStruct((E, 4), jnp.float32),
    )(f1, mt, w2, b2.reshape(1, 64), w3, b3.reshape(1, 4))


def _sc_msg_call(hm5, row, bond, col, zeros2d):
    mesh = plsc.VectorSubcoreMesh(core_axis_name="c", subcore_axis_name="s")
    out_type = jax.ShapeDtypeStruct((2 * N, 128), jnp.float32)

    def body(hm5_h, row_h, bond_h, col_h, z2_h, agg_h,
             rowv, bondv, colv1, keyv, colv, rows, acc, semi, semg, sems):
        c = lax.axis_index("c")
        s = lax.axis_index("s")
        r0_acc = pl.multiple_of(s * _RPT, 8)
        pltpu.sync_copy(z2_h.at[pl.ds(r0_acc, _RPT)],
                        acc.at[pl.ds(r0_acc, _RPT)])
        plsc.subcore_barrier()

        nch = jnp.where(s < _EXTRA, _TCH // 16 + 1, _TCH // 16)

        def chunk(k, carry):
            base = pl.multiple_of((s + 16 * k) * _CH, 8)
            c1 = pltpu.async_copy(row_h.at[pl.ds(base, _CH)], rowv, semi)
            c2 = pltpu.async_copy(bond_h.at[pl.ds(base, _CH)], bondv, semi)
            c3 = pltpu.async_copy(col_h.at[pl.ds(base, _CH)], colv1, semi)
            c1.wait(); c2.wait(); c3.wait()
            cN = c * N
            for j in range(_NSUB):
                for m in range(_SUB // 16):
                    sl = pl.ds(j * _SUB + m * 16, 16)
                    keyv[j, pl.ds(m * 16, 16)] = (
                        bondv[sl] * (2 * N) + rowv[sl] + cN)
                    colv[j, pl.ds(m * 16, 16)] = colv1[sl]
            gs = [pltpu.async_copy(hm5_h.at[keyv.at[j]],
                                   rows.at[pl.ds(j * _SUB, _SUB)], semg)
                  for j in range(_NSUB)]
            for g in gs:
                g.wait()
            ss = [pltpu.async_copy(rows.at[pl.ds(j * _SUB, _SUB)],
                                   acc.at[colv.at[j]], sems, add=True)
                  for j in range(_NSUB)]
            for t in ss:
                t.wait()
            return carry

        lax.fori_loop(0, nch, chunk, 0)
        plsc.subcore_barrier()

        _LAST = N - 15 * _RPT

        @pl.when(s < 15)
        def _():
            pltpu.sync_copy(
                acc.at[pl.ds(r0_acc, _RPT)],
                agg_h.at[pl.ds(pl.multiple_of(c * N + r0_acc, 8), _RPT)])

        @pl.when(s == 15)
        def _():
            pltpu.sync_copy(
                acc.at[pl.ds(15 * _RPT, _LAST)],
                agg_h.at[pl.ds(pl.multiple_of(c * N + 15 * _RPT, 8), _LAST)])

    scratch = [
        pltpu.VMEM((_CH,), jnp.int32),
        pltpu.VMEM((_CH,), jnp.int32),
        pltpu.VMEM((_CH,), jnp.int32),
        pltpu.VMEM((_NSUB, _SUB), jnp.int32),
        pltpu.VMEM((_NSUB, _SUB), jnp.int32),
        pltpu.VMEM((_CH, 128), jnp.float32),
        pltpu.VMEM_SHARED((NPAD, 128), jnp.float32),
        pltpu.SemaphoreType.DMA,
        pltpu.SemaphoreType.DMA,
        pltpu.SemaphoreType.DMA,
    ]
    fn = pl.kernel(body, out_type=out_type, mesh=mesh, scratch_types=scratch)
    return fn(hm5, row, bond, col, zeros2d)


_CCH = 160
_CTCH = E // _CCH
_CEXTRA = _CTCH - 32 * (_CTCH // 32)


def _sc_edge_gather_call(an, bn, row, col):
    mesh = plsc.VectorSubcoreMesh(core_axis_name="c", subcore_axis_name="s")
    out_type = (jax.ShapeDtypeStruct((E, 128), jnp.float32),
                jax.ShapeDtypeStruct((E, 32), jnp.float32))

    def body(an_h, bn_h, row_h, col_h, f1_h, mt_h,
             ridx, cidx, bufa, bufb, obuf, mbuf, semi, semg, semo):
        c = lax.axis_index("c")
        s = lax.axis_index("s")
        wid = c * 16 + s
        nch = jnp.where(wid < _CEXTRA, _CTCH // 32 + 1, _CTCH // 32)

        def chunk(k, carry):
            base = pl.multiple_of((wid + 32 * k) * _CCH, 8)
            c1 = pltpu.async_copy(row_h.at[pl.ds(base, _CCH)], ridx, semi)
            c2 = pltpu.async_copy(col_h.at[pl.ds(base, _CCH)], cidx, semi)
            c1.wait(); c2.wait()
            g1 = pltpu.async_copy(an_h.at[ridx], bufa, semg)
            g2 = pltpu.async_copy(bn_h.at[cidx], bufb, semg)
            g1.wait(); g2.wait()

            def edge(e, cc):
                for g in range(8):
                    sl = pl.ds(g * 16, 16)
                    obuf[e, sl] = jnp.maximum(bufa[e, sl] + bufb[e, sl], 0.0)
                mbuf[e, pl.ds(0, 16)] = bufa[e, pl.ds(128, 16)]
                mbuf[e, pl.ds(16, 16)] = bufb[e, pl.ds(128, 16)]
                return cc

            lax.fori_loop(0, _CCH, edge, 0)
            o1 = pltpu.async_copy(obuf, f1_h.at[pl.ds(base, _CCH)], semo)
            o2 = pltpu.async_copy(mbuf, mt_h.at[pl.ds(base, _CCH)], semo)
            o1.wait(); o2.wait()
            return carry

        lax.fori_loop(0, nch, chunk, 0)

    scratch = [
        pltpu.VMEM((_CCH,), jnp.int32),
        pltpu.VMEM((_CCH,), jnp.int32),
        pltpu.VMEM((_CCH, 256), jnp.float32),
        pltpu.VMEM((_CCH, 256), jnp.float32),
        pltpu.VMEM((_CCH, 128), jnp.float32),
        pltpu.VMEM((_CCH, 32), jnp.float32),
        pltpu.SemaphoreType.DMA,
        pltpu.SemaphoreType.DMA,
        pltpu.SemaphoreType.DMA,
    ]
    fn = pl.kernel(body, out_type=out_type, mesh=mesh, scratch_types=scratch)
    return fn(an, bn, row, col)


def kernel(x, edge_index, edge_attr, batch, params):
    p = params
    at = jnp.clip(x[:, 0].astype(jnp.int32), 0, 10)
    bond = jnp.clip(edge_attr[:, 0].astype(jnp.int32), 0, 4)
    row = edge_index[0].astype(jnp.int32)
    col = edge_index[1].astype(jnp.int32)

    atom_table = p['atom_table']
    h1t = jnp.maximum(atom_table @ p['vp_w1'] + p['vp_b1'], 0.0)
    vlt = h1t @ p['vp_w2'] + p['vp_b2']
    pv_tab = jnp.argmax(vlt, axis=-1).astype(jnp.int32) + 1
    h0_tab = jnp.concatenate([
        atom_table,
        jax.nn.one_hot(pv_tab - 1, 8, dtype=jnp.float32),
        jnp.zeros((11, H - 72), jnp.float32),
    ], axis=-1)

    wm_h = [p['g%d_wm' % l][:H] for l in range(3)]
    wm_b = [p['g%d_wm' % l][H:] for l in range(3)]
    btm = [p['bond_table'] @ wm_b[l] for l in range(3)]
    hm0_tab = h0_tab @ wm_h[0] + p['g0_bm']

    pv_tab_f = pv_tab.astype(jnp.float32)[:, None]
    big = jnp.concatenate([
        h0_tab, hm0_tab, vlt, pv_tab_f, pv_tab_f / 8.0,
        jnp.zeros((11, 640 - 522), jnp.float32)], axis=1)
    big = jnp.concatenate([big, jnp.zeros((5, 640), jnp.float32)], axis=0)

    nodevals = _expand(at[:, None], big)
    h = nodevals[:, :256]
    hm = nodevals[:, 256:512]
    valence_logits = nodevals[:, 512:520]
    pv_f = nodevals[:, 520:521]
    gate = nodevals[:, 521:522]

    zeros2d = jnp.zeros((NPAD, 128), jnp.float32)

    degree = jnp.zeros((N,), jnp.float32).at[col].add(1.0)
    for l in range(3):
        hm5 = _hm5(hm, btm[l])
        agg2 = _sc_msg_call(hm5, row, bond, col, zeros2d)
        wu = p['g%d_wu' % l]
        h = _gnn_update(h, agg2[:N], agg2[N:], gate,
                        wu[:H], wu[H:H + 128], wu[H + 128:],
                        p['g%d_bu' % l])
        if l < 2:
            hm = _mm_bias(h, wm_h[l + 1], p['g%d_bm' % (l + 1)])

    w1 = p['btc_w1']
    at_f = at.astype(jnp.float32)
    hf, anode, bnode, props, vio = _epilogue(
        h, pv_f, at_f[:, None], degree[:, None],
        w1[:H], w1[H:2 * H], w1[2 * H:], p['btc_b1'],
        p['cp_w1'], p['cp_b1'], p['cp_w2'], p['cp_b2'])
    violations = vio[:, 0]

    f1, meta = _sc_edge_gather_call(anode, bnode, row, col)
    bond_logits = _edge_cls(f1, meta,
                            p['btc_w2'], p['btc_b2'],
                            p['btc_w3'], p['btc_b3'])

    return (hf, props, valence_logits, bond_logits, violations)
